# baseline (speedup 1.0000x reference)
import functools

import jax
import jax.numpy as jnp
from jax import lax
from jax.experimental import pallas as pl
from jax.experimental.pallas import tpu as pltpu
from jax.experimental.pallas import tpu_sc as plsc

N_NODES = 10000
N_EDGES = 320000
CPAD = 10240
CROWS = 80
CCOLS = 128
NSUB = 16
NCORE = 2
TILES = NSUB * NCORE
EPT = N_EDGES // TILES
NVEC = EPT // 16
RPS = CROWS // NSUB
IDX_SHIFT = 7
IDX_MASK = 127


def _sc_edge_passes(edge_index_flat):
    mesh = plsc.VectorSubcoreMesh(core_axis_name="c", subcore_axis_name="s",
                                  num_cores=NCORE)

    @functools.partial(
        pl.kernel,
        mesh=mesh,
        compiler_params=pltpu.CompilerParams(needs_layout_passes=False),
        out_type=[
            jax.ShapeDtypeStruct((8, CROWS, CCOLS), jnp.float32),
            jax.ShapeDtypeStruct((4 * CROWS, CCOLS), jnp.float32),
            jax.ShapeDtypeStruct((64,), jnp.int32),
        ],
        scratch_types=[
            pltpu.VMEM((EPT,), jnp.int32),
            pltpu.VMEM((EPT,), jnp.int32),
            pltpu.VMEM((CROWS, CCOLS), jnp.float32),
            pltpu.VMEM((CROWS, CCOLS), jnp.float32),
            pltpu.VMEM((CROWS, CCOLS), jnp.float32),
            pltpu.VMEM((CROWS, CCOLS), jnp.float32),
            pltpu.VMEM((RPS, CCOLS), jnp.float32),
            pltpu.VMEM((RPS, CCOLS), jnp.float32),
            pltpu.VMEM((16,), jnp.int32),
            pltpu.VMEM((16,), jnp.int32),
            pltpu.VMEM((16,), jnp.int32),
            pltpu.VMEM_SHARED((NSUB, CROWS, CCOLS), jnp.float32),
            pltpu.VMEM_SHARED((CROWS, CCOLS), jnp.float32),
        ],
    )
    def k(ei_hbm, wmat_hbm, xch_hbm, flg_hbm,
          src_v, dst_v, priv, fullA, fullB, oth, sacc, stmp,
          nbuf, flv, oflv, partials, comb):
        cid = lax.axis_index("c")
        sid = lax.axis_index("s")
        wid = cid * NSUB + sid
        is_writer = sid == 0
        r0 = sid * RPS
        other = 1 - cid

        pltpu.sync_copy(ei_hbm.at[pl.ds(wid * EPT, EPT)], src_v)
        pltpu.sync_copy(ei_hbm.at[pl.ds(N_EDGES + wid * EPT, EPT)], dst_v)
        pltpu.sync_copy(ei_hbm.at[pl.ds(0, 16)], nbuf)

        def zero_tab(tab):
            def row(r, _):
                for kk in range(CCOLS // 16):
                    tab[r, pl.ds(kk * 16, 16)] = jnp.zeros((16,), jnp.float32)
                return 0
            lax.fori_loop(0, CROWS, row, 0)

        def expected_flag(kpass):
            mix = jnp.int32((0x3C6EF372 * (kpass + 1)) & 0x7FFFFFFF)
            return lax.bitwise_xor(nbuf[pl.ds(0, 16)], mix)

        def combine(kpass, exchange):
            pltpu.sync_copy(priv, partials.at[sid])
            plsc.subcore_barrier()
            for r in range(RPS):
                for kk in range(CCOLS // 16):
                    sacc[r, pl.ds(kk * 16, 16)] = jnp.zeros((16,), jnp.float32)

            def red_body(p, _):
                pltpu.sync_copy(partials.at[p, pl.ds(r0, RPS)], stmp)
                for r in range(RPS):
                    for kk in range(CCOLS // 16):
                        s = pl.ds(kk * 16, 16)
                        sacc[r, s] = sacc[r, s] + stmp[r, s]
                return 0

            lax.fori_loop(0, NSUB, red_body, 0)
            pltpu.sync_copy(sacc, comb.at[pl.ds(r0, RPS)])
            plsc.subcore_barrier()
            if exchange:
                exp = expected_flag(kpass)

                @pl.when(is_writer)
                def _():
                    base_own = (kpass * NCORE + cid) * CROWS
                    pltpu.sync_copy(comb, xch_hbm.at[pl.ds(base_own, CROWS)])
                    flv[pl.ds(0, 16)] = exp
                    pltpu.sync_copy(
                        flv, flg_hbm.at[pl.ds((kpass * NCORE + cid) * 16, 16)])

                    def cond(ok):
                        return jnp.logical_not(ok)

                    def body(ok):
                        pltpu.sync_copy(
                            flg_hbm.at[pl.ds((kpass * NCORE + other) * 16, 16)],
                            oflv)
                        return jnp.all(oflv[pl.ds(0, 16)] == exp)

                    lax.while_loop(cond, body, jnp.bool_(False))

                plsc.subcore_barrier()
                base_oth = (kpass * NCORE) * CROWS + other * CROWS
                pltpu.sync_copy(xch_hbm.at[pl.ds(base_oth, CROWS)], oth)

        zero_tab(priv)

        @pl.when(is_writer & (cid == 0))
        def _():
            def ones_row(r, _):
                base = r * CCOLS
                for kk in range(CCOLS // 16):
                    gidx = lax.iota(jnp.int32, 16) + (base + kk * 16)
                    fullB[r, pl.ds(kk * 16, 16)] = jnp.where(
                        gidx < N_NODES, 1.0, 0.0).astype(jnp.float32)
                return 0

            lax.fori_loop(0, CROWS, ones_row, 0)
            pltpu.sync_copy(fullB, wmat_hbm.at[0])
            for rr in range(5, 8):
                pltpu.sync_copy(priv, wmat_hbm.at[rr])

        @plsc.parallel_loop(0, NVEC, unroll=4)
        def _(i):
            d16 = dst_v[pl.ds(i * 16, 16)]
            hi = lax.shift_right_logical(d16, IDX_SHIFT)
            lo = lax.bitwise_and(d16, IDX_MASK)
            plsc.addupdate_scatter(priv, [hi, lo],
                                   jnp.ones((16,), jnp.float32))

        combine(0, exchange=True)
        pltpu.sync_copy(comb, fullA)

        def invc_row(r, _):
            for kk in range(CCOLS // 16):
                s = pl.ds(kk * 16, 16)
                fullA[r, s] = 1.0 / jnp.maximum(fullA[r, s] + oth[r, s], 1.0)
            return 0

        lax.fori_loop(0, CROWS, invc_row, 0)

        zero_tab(priv)

        def run_gather_pass(gather_tab):
            @plsc.parallel_loop(0, NVEC, unroll=4)
            def _(i):
                d16 = dst_v[pl.ds(i * 16, 16)]
                s16 = src_v[pl.ds(i * 16, 16)]
                ghi = lax.shift_right_logical(d16, IDX_SHIFT)
                glo = lax.bitwise_and(d16, IDX_MASK)
                wv = plsc.load_gather(gather_tab, [ghi, glo])
                shi = lax.shift_right_logical(s16, IDX_SHIFT)
                slo = lax.bitwise_and(s16, IDX_MASK)
                plsc.addupdate_scatter(priv, [shi, slo], wv)

        run_gather_pass(fullA)
        combine(1, exchange=True)
        pltpu.sync_copy(comb, fullB)

        @pl.when(is_writer)
        def _():
            pltpu.sync_copy(comb, wmat_hbm.at[1 + cid])

        def q_row(r, _):
            for kk in range(CCOLS // 16):
                s = pl.ds(kk * 16, 16)
                fullB[r, s] = (fullB[r, s] + oth[r, s]) * fullA[r, s]
            return 0

        lax.fori_loop(0, CROWS, q_row, 0)

        zero_tab(priv)
        run_gather_pass(fullB)
        combine(2, exchange=False)

        @pl.when(is_writer)
        def _():
            pltpu.sync_copy(comb, wmat_hbm.at[3 + cid])

        plsc.subcore_barrier()

    return k(edge_index_flat)


def _tc_reduce_mlp(x, wmat, W1_l, b1_l, W1_r, W2_l, b2_l, W2_r,
                   fc1_W, fc1_b, fc2_Wp, fc2_bp):
    def dgT(a, W):
        return lax.dot_general(a, W, (((1,), (1,)), ((), ())),
                               preferred_element_type=jnp.float32)

    def body(xb, wb, W1_l_r, b1_l_r, W1_r_r, W2_l_r, b2_l_r, W2_r_r,
             fc1_W_r, fc1_b_r, fc2_W_r, fc2_b_r, out_ref):
        acc = jnp.dot(wb[...], xb[...], preferred_element_type=jnp.float32)
        u2 = acc[0:1, :]
        v2 = acc[1:2, :] + acc[2:3, :]
        v1 = acc[3:4, :] + acc[4:5, :]
        sumw = jnp.sum(wb[1:3, :])
        n = jnp.float32(N_NODES)
        S1h = dgT(v2, W1_l_r[...]) + n * b1_l_r[...] + dgT(u2, W1_r_r[...])
        S2 = dgT(v1, W1_l_r[...]) + sumw * b1_l_r[...] + dgT(v2, W1_r_r[...])
        me = (dgT(S2, W2_l_r[...]) + n * b2_l_r[...]
              + dgT(S1h, W2_r_r[...])) * (1.0 / n)
        hid = jnp.maximum(dgT(me, fc1_W_r[...]) + fc1_b_r[...], 0.0)
        qp = dgT(hid, fc2_W_r[...]) + fc2_b_r[...]
        out_ref[...] = jnp.broadcast_to(qp, (8, 128))

    return pl.pallas_call(
        body,
        out_shape=jax.ShapeDtypeStruct((8, 128), jnp.float32),
    )(x, wmat, W1_l, b1_l, W1_r, W2_l, b2_l, W2_r,
      fc1_W, fc1_b, fc2_Wp, fc2_bp)


def kernel(x, edge_index, W1_l, b1_l, W1_r, W2_l, b2_l, W2_r,
           fc1_W, fc1_b, fc2_W, fc2_b):
    ei_flat = edge_index.astype(jnp.int32).reshape(2 * N_EDGES)
    wmat3d, _, _ = _sc_edge_passes(ei_flat)
    wmat = wmat3d.reshape(8, CPAD)[:, :N_NODES]

    fc2_Wp = jnp.pad(fc2_W, ((0, 128 - fc2_W.shape[0]), (0, 0)))
    fc2_bp = jnp.pad(fc2_b, (0, 128 - fc2_b.shape[0])).reshape(1, 128)

    out = _tc_reduce_mlp(
        x, wmat,
        W1_l, b1_l.reshape(1, 128), W1_r,
        W2_l, b2_l.reshape(1, 128), W2_r,
        fc1_W, fc1_b.reshape(1, 256), fc2_Wp, fc2_bp)
    return out[0, :100]


# speedup vs baseline: 1.0405x; 1.0405x over previous
import functools

import jax
import jax.numpy as jnp
from jax import lax
from jax.experimental import pallas as pl
from jax.experimental.pallas import tpu as pltpu
from jax.experimental.pallas import tpu_sc as plsc

N_NODES = 10000
N_EDGES = 320000
CPAD = 10240
CROWS = 80
CCOLS = 128
NSUB = 16
EPT = N_EDGES // NSUB
NVEC = EPT // 16
RPS = CROWS // NSUB
IDX_SHIFT = 7
IDX_MASK = 127


def _dedup_scatter_add(priv, skey, cs, nxt):
    lane = lax.iota(jnp.int32, 16)
    is_last = (skey != nxt) | (lane == 15)
    hi = lax.shift_right_logical(skey, IDX_SHIFT)
    lo = lax.bitwise_and(skey, IDX_MASK)
    plsc.addupdate_scatter(priv, [hi, lo], cs, mask=is_last)
    m2 = is_last & (lane < 15)
    nhi = lax.shift_right_logical(nxt, IDX_SHIFT)
    nlo = lax.bitwise_and(nxt, IDX_MASK)
    plsc.addupdate_scatter(priv, [nhi, nlo], -cs, mask=m2)


def _shift_up(v):
    perm = jnp.minimum(lax.iota(jnp.int32, 16) + 1, 15)
    return v.at[perm].get(mode="promise_in_bounds")


def _table_map(tab, fn):
    def row(r, _):
        for kk in range(CCOLS // 16):
            tab[r, pl.ds(kk * 16, 16)] = fn(tab[r, pl.ds(kk * 16, 16)])
        return 0
    lax.fori_loop(0, CROWS, row, 0)


def _sc_edge_passes(edge_index):
    mesh = plsc.VectorSubcoreMesh(core_axis_name="c", subcore_axis_name="s",
                                  num_cores=1)

    @functools.partial(
        pl.kernel,
        mesh=mesh,
        compiler_params=pltpu.CompilerParams(needs_layout_passes=False),
        out_type=jax.ShapeDtypeStruct((8, CROWS, CCOLS), jnp.float32),
        scratch_types=[
            pltpu.VMEM((EPT,), jnp.int32),
            pltpu.VMEM((EPT,), jnp.int32),
            pltpu.VMEM((CROWS, CCOLS), jnp.float32),
            pltpu.VMEM((CROWS, CCOLS), jnp.float32),
            pltpu.VMEM((CROWS, CCOLS), jnp.float32),
            pltpu.VMEM((RPS, CCOLS), jnp.float32),
            pltpu.VMEM((RPS, CCOLS), jnp.float32),
            pltpu.SemaphoreType.DMA,
            pltpu.VMEM_SHARED((NSUB, CROWS, CCOLS), jnp.float32),
            pltpu.VMEM_SHARED((CROWS, CCOLS), jnp.float32),
        ],
    )
    def k(ei_hbm, wmat_hbm,
          src_v, dst_v, priv, fullA, fullB, sacc, stmp16, ssem,
          partials, comb):
        sid = lax.axis_index("s")
        is_writer = sid == 0
        r0 = sid * RPS

        src_cp = pltpu.async_copy(ei_hbm.at[pl.ds(sid * EPT, EPT)], src_v,
                                  ssem)
        pltpu.sync_copy(ei_hbm.at[pl.ds(N_EDGES + sid * EPT, EPT)], dst_v)

        def begin_pass():
            _table_map(priv, lambda v: jnp.zeros((16,), jnp.float32))

        def combine():
            pltpu.sync_copy(priv, partials.at[sid])
            plsc.subcore_barrier()
            for r in range(RPS):
                for kk in range(CCOLS // 16):
                    sacc[r, pl.ds(kk * 16, 16)] = jnp.zeros((16,), jnp.float32)

            def red_body(p, _):
                pltpu.sync_copy(partials.at[p, pl.ds(r0, RPS)], stmp16)
                for r in range(RPS):
                    for kk in range(CCOLS // 16):
                        s = pl.ds(kk * 16, 16)
                        sacc[r, s] = sacc[r, s] + stmp16[r, s]
                return 0

            lax.fori_loop(0, NSUB, red_body, 0)
            pltpu.sync_copy(sacc, comb.at[pl.ds(r0, RPS)])
            plsc.subcore_barrier()

        begin_pass()

        @pl.when(is_writer)
        def _():
            def ones_row(r, _):
                base = r * CCOLS
                for kk in range(CCOLS // 16):
                    gidx = lax.iota(jnp.int32, 16) + (base + kk * 16)
                    fullB[r, pl.ds(kk * 16, 16)] = jnp.where(
                        gidx < N_NODES, 1.0, 0.0).astype(jnp.float32)
                return 0

            lax.fori_loop(0, CROWS, ones_row, 0)
            pltpu.sync_copy(fullB, wmat_hbm.at[0])
            for rr in range(3, 8):
                pltpu.sync_copy(priv, wmat_hbm.at[rr])

        @plsc.parallel_loop(0, NVEC, unroll=4)
        def _(i):
            d16 = dst_v[pl.ds(i * 16, 16)]
            hi = lax.shift_right_logical(d16, IDX_SHIFT)
            lo = lax.bitwise_and(d16, IDX_MASK)
            plsc.addupdate_scatter(priv, [hi, lo],
                                   jnp.ones((16,), jnp.float32))
        combine()
        pltpu.sync_copy(comb, fullA)
        plsc.subcore_barrier()

        _table_map(fullA, lambda v: 1.0 / jnp.maximum(v, 1.0))

        begin_pass()

        def run_p23(gather_tab):
            @plsc.parallel_loop(0, NVEC, unroll=4)
            def _(i):
                d16 = dst_v[pl.ds(i * 16, 16)]
                s16 = src_v[pl.ds(i * 16, 16)]
                ghi = lax.shift_right_logical(d16, IDX_SHIFT)
                glo = lax.bitwise_and(d16, IDX_MASK)
                wv = plsc.load_gather(gather_tab, [ghi, glo])
                shi = lax.shift_right_logical(s16, IDX_SHIFT)
                slo = lax.bitwise_and(s16, IDX_MASK)
                plsc.addupdate_scatter(priv, [shi, slo], wv)

        src_cp.wait()
        run_p23(fullA)
        combine()
        pltpu.sync_copy(comb, fullB)

        @pl.when(is_writer)
        def _():
            pltpu.sync_copy(comb, wmat_hbm.at[1])

        plsc.subcore_barrier()

        def q_row(r, _):
            for kk in range(CCOLS // 16):
                s = pl.ds(kk * 16, 16)
                fullB[r, s] = fullB[r, s] * fullA[r, s]
            return 0

        lax.fori_loop(0, CROWS, q_row, 0)

        begin_pass()
        run_p23(fullB)
        combine()

        @pl.when(is_writer)
        def _():
            pltpu.sync_copy(comb, wmat_hbm.at[2])

        plsc.subcore_barrier()

    return k(edge_index)


def _tc_reduce_mlp(xp, wmat, W1_l, b1_l, W1_r, W2_l, b2_l, W2_r,
                   fc1_W, fc1_b, fc2_Wp, fc2_bp):
    def dgT(a, W):
        return lax.dot_general(a, W, (((1,), (1,)), ((), ())),
                               preferred_element_type=jnp.float32)

    def body(xb, wb, W1_l_r, b1_l_r, W1_r_r, W2_l_r, b2_l_r, W2_r_r,
             fc1_W_r, fc1_b_r, fc2_W_r, fc2_b_r, out_ref):
        acc = jnp.dot(wb[...], xb[...], preferred_element_type=jnp.float32)
        u2 = acc[0:1, :]
        v2 = acc[1:2, :]
        v1 = acc[2:3, :]
        sumw = jnp.sum(wb[1, :])
        n = jnp.float32(N_NODES)
        S1h = dgT(v2, W1_l_r[...]) + n * b1_l_r[...] + dgT(u2, W1_r_r[...])
        S2 = dgT(v1, W1_l_r[...]) + sumw * b1_l_r[...] + dgT(v2, W1_r_r[...])
        me = (dgT(S2, W2_l_r[...]) + n * b2_l_r[...]
              + dgT(S1h, W2_r_r[...])) * (1.0 / n)
        hid = jnp.maximum(dgT(me, fc1_W_r[...]) + fc1_b_r[...], 0.0)
        qp = dgT(hid, fc2_W_r[...]) + fc2_b_r[...]
        out_ref[...] = jnp.broadcast_to(qp, (8, 128))

    return pl.pallas_call(
        body,
        out_shape=jax.ShapeDtypeStruct((8, 128), jnp.float32),
    )(xp, wmat, W1_l, b1_l, W1_r, W2_l, b2_l, W2_r,
      fc1_W, fc1_b, fc2_Wp, fc2_bp)


def kernel(x, edge_index, W1_l, b1_l, W1_r, W2_l, b2_l, W2_r,
           fc1_W, fc1_b, fc2_W, fc2_b):
    ei_flat = edge_index.astype(jnp.int32).reshape(2 * N_EDGES)
    wmat = _sc_edge_passes(ei_flat).reshape(8, CPAD)[:, :N_NODES]

    fc2_Wp = jnp.pad(fc2_W, ((0, 128 - fc2_W.shape[0]), (0, 0)))
    fc2_bp = jnp.pad(fc2_b, (0, 128 - fc2_b.shape[0])).reshape(1, 128)

    out = _tc_reduce_mlp(
        x, wmat,
        W1_l, b1_l.reshape(1, 128), W1_r,
        W2_l, b2_l.reshape(1, 128), W2_r,
        fc1_W, fc1_b.reshape(1, 256), fc2_Wp, fc2_bp)
    return out[0, :100]
